# baseline (speedup 1.0000x reference)
import jax
import jax.numpy as jnp
from jax.experimental import pallas as pl
from jax.experimental.pallas import tpu as pltpu

_TB = 1024


def _router_kernel(x_ref, r_ref, eg_ref, o_ref):
    eg = eg_ref[...]
    e_dim = eg.shape[0]
    dots = jax.lax.dot_general(
        eg, x_ref[...], (((1,), (1,)), ((), ())),
        preferred_element_type=jnp.float32,
    )
    scores = dots + jnp.transpose(r_ref[...])

    row = jax.lax.broadcasted_iota(jnp.int32, scores.shape, 0)
    m1 = jnp.max(scores, axis=0, keepdims=True)
    i1 = jnp.min(jnp.where(scores == m1, row, e_dim), axis=0, keepdims=True)
    masked = jnp.where(row == i1, -jnp.inf, scores)
    m2 = jnp.max(masked, axis=0, keepdims=True)
    i2 = jnp.min(jnp.where(masked == m2, row, e_dim), axis=0, keepdims=True)
    e = jnp.exp(m2 - m1)
    ssum = 1.0 + e
    o_ref[0:1, :] = i1.astype(jnp.float32)
    o_ref[1:2, :] = 1.0 / ssum
    o_ref[2:3, :] = i2.astype(jnp.float32)
    o_ref[3:4, :] = e / ssum


def kernel(token_hidden, router_logits, expert_ground, alpha):
    T, D = token_hidden.shape
    E = expert_ground.shape[0]
    eg = jnp.float32(alpha) * expert_ground

    out = pl.pallas_call(
        _router_kernel,
        grid=(T // _TB,),
        in_specs=[
            pl.BlockSpec((_TB, D), lambda i: (i, 0)),
            pl.BlockSpec((_TB, E), lambda i: (i, 0)),
            pl.BlockSpec((E, D), lambda i: (0, 0)),
        ],
        out_specs=pl.BlockSpec((4, _TB), lambda i: (0, i)),
        out_shape=jax.ShapeDtypeStruct((4, T), jnp.float32),
        compiler_params=pltpu.CompilerParams(
            dimension_semantics=("parallel",),
        ),
    )(token_hidden, router_logits, eg)

    return out.T.reshape(T, 2, 2)


# speedup vs baseline: 1.0051x; 1.0051x over previous
import jax
import jax.numpy as jnp
from jax.experimental import pallas as pl
from jax.experimental.pallas import tpu as pltpu

_TB = 1024


def _router_kernel(x_ref, r_ref, eg_ref, o_ref):
    eg = eg_ref[...]
    e_dim = eg.shape[0]
    dots = jax.lax.dot_general(
        eg, x_ref[...], (((1,), (1,)), ((), ())),
        preferred_element_type=jnp.float32,
    )
    scores = dots + jnp.transpose(r_ref[...])

    row = jax.lax.broadcasted_iota(jnp.int32, scores.shape, 0)
    m1 = jnp.max(scores, axis=0, keepdims=True)
    i1 = jnp.min(jnp.where(scores == m1, row, e_dim), axis=0, keepdims=True)
    masked = jnp.where(row == i1, -jnp.inf, scores)
    m2 = jnp.max(masked, axis=0, keepdims=True)
    i2 = jnp.min(jnp.where(masked == m2, row, e_dim), axis=0, keepdims=True)
    e = jnp.exp(m2 - m1)
    ssum = 1.0 + e
    o_ref[0:1, :] = i1.astype(jnp.float32)
    o_ref[1:2, :] = 1.0 / ssum
    o_ref[2:3, :] = i2.astype(jnp.float32)
    o_ref[3:4, :] = e / ssum


def kernel(token_hidden, router_logits, expert_ground, alpha):
    T, D = token_hidden.shape
    E = expert_ground.shape[0]
    eg = jnp.float32(alpha) * expert_ground

    out = pl.pallas_call(
        _router_kernel,
        grid=(T // _TB,),
        in_specs=[
            pl.BlockSpec((_TB, D), lambda i: (i, 0)),
            pl.BlockSpec((_TB, E), lambda i: (i, 0)),
            pl.BlockSpec((E, D), lambda i: (0, 0)),
        ],
        out_specs=pl.BlockSpec((4, _TB), lambda i: (0, i)),
        out_shape=jax.ShapeDtypeStruct((4, T), jnp.float32),
        compiler_params=pltpu.CompilerParams(
            dimension_semantics=("arbitrary",),
        ),
    )(token_hidden, router_logits, eg)

    return out.T.reshape(T, 2, 2)
